# baseline (speedup 1.0000x reference)
import functools

import jax
import jax.numpy as jnp
from jax import lax
from jax.experimental import pallas as pl
from jax.experimental.pallas import tpu as pltpu
from jax.experimental.pallas import tpu_sc as plsc

B, N, M, C, NB, L = 4, 2048, 48, 32, 16, 2
RC = 5.0
E = B * N * M
BN = B * N
TN = 128
TNM = TN * M
NBLK = BN // TN
ER = E // 128
GR = ER // NBLK
NG = NB + 4

NC, NS = 2, 16
NW = NC * NS
PW = E // NW
G = 128
CH = PW // G


def _sc_mesh():
    return plsc.VectorSubcoreMesh(core_axis_name="c", subcore_axis_name="s")


def _make_static_gather():
    @functools.partial(
        pl.kernel,
        mesh=_sc_mesh(),
        out_type=jax.ShapeDtypeStruct((4, E), jnp.float32),
        compiler_params=pltpu.CompilerParams(needs_layout_passes=False),
        scratch_types=[
            pltpu.VMEM((4 * BN,), jnp.float32),
            pltpu.VMEM((PW,), jnp.int32),
            pltpu.VMEM((4, PW), jnp.float32),
            pltpu.SemaphoreType.DMA,
        ],
    )
    def static_gather(tab_hbm, idx_hbm, out_hbm, tab_v, idx_v, out_v, sem):
        wid = lax.axis_index("s") * NC + lax.axis_index("c")
        base = wid * PW
        pltpu.sync_copy(tab_hbm, tab_v)
        pltpu.sync_copy(idx_hbm.at[pl.ds(base, PW)], idx_v)

        def body(i, carry):
            iv = idx_v[pl.ds(i * 16, 16)]
            for p in range(4):
                out_v[p, pl.ds(i * 16, 16)] = plsc.load_gather(
                    tab_v, [iv + p * BN])
            return carry

        lax.fori_loop(0, PW // 16, body, 0)
        for p in range(4):
            pltpu.sync_copy(out_v.at[p], out_hbm.at[p, pl.ds(base, PW)])

    return static_gather


def _make_row_gather(D):
    @functools.partial(
        pl.kernel,
        mesh=_sc_mesh(),
        out_type=jax.ShapeDtypeStruct((E, D), jnp.float32),
        compiler_params=pltpu.CompilerParams(use_tc_tiling_on_sc=False),
        scratch_types=[
            pltpu.VMEM((CH, G), jnp.int32),
            pltpu.VMEM((G, D), jnp.float32),
            pltpu.SemaphoreType.DMA,
        ],
    )
    def row_gather(table_hbm, idx_hbm, out_hbm, idx_v, rows_v, sem):
        wid = lax.axis_index("s") * NC + lax.axis_index("c")
        base = wid * PW
        pltpu.sync_copy(idx_hbm.at[wid], idx_v)

        def body(j, carry):
            pltpu.async_copy(table_hbm.at[idx_v.at[j]], rows_v, sem).wait()
            pltpu.sync_copy(rows_v, out_hbm.at[pl.ds(base + j * G, G)])
            return carry

        lax.fori_loop(0, CH, body, 0)

    return row_gather


_sc_cache = {}


def _get_sc(name):
    if name not in _sc_cache:
        _sc_cache[name] = (_make_static_gather() if name == "static"
                           else _make_row_gather(2 * C))
    return _sc_cache[name]


def _silu(x):
    return x * (1.0 / (1.0 + jnp.exp(-x)))


def _geo_kernel(e4_ref, cen_ref, mask_ref, geo_ref):
    rx = e4_ref[0] - cen_ref[0]
    ry = e4_ref[1] - cen_ref[1]
    rz = e4_ref[2] - cen_ref[2]
    d = jnp.sqrt(rx * rx + ry * ry + rz * rz + 1e-12)
    inv = 1.0 / d
    geo_ref[0] = rx * inv
    geo_ref[1] = ry * inv
    geo_ref[2] = rz * inv
    dc = jnp.clip(d, 0.0, RC)
    env = 0.5 * (jnp.cos(dc * (jnp.pi / RC)) + 1.0) * (d < RC).astype(jnp.float32)
    maskf = mask_ref[0] * env
    geo_ref[NB + 3] = maskf
    th = d * (jnp.pi / RC)
    sinv = inv * maskf
    for k in range(NB):
        geo_ref[3 + k] = jnp.sin((k + 1.0) * th) * sinv


def _seg_sum128(x):
    return jnp.sum(x.reshape(TN, M, 4 * C), axis=1)


_TDN = (((0,), (0,)), ((), ()))


def _layer0_kernel(geo_ref, zj_ref, zc_ref, emb4_ref, cls_ref, p4_ref,
                   wrb_ref, w0_ref, u0_ref, b0_ref,
                   w1_ref, wg_ref, bg_ref, hc_ref, hp_ref):
    g = geo_ref[...]
    filt = jnp.dot(g[3:3 + NB + 1].T, wrb_ref[...],
                   preferred_element_type=jnp.float32)
    oh = (cls_ref[...] == zj_ref[...]).astype(jnp.float32)
    h0j4 = jnp.dot(oh, emb4_ref[...],
                   preferred_element_type=jnp.float32)
    uaug = jnp.concatenate(
        [jnp.ones((1, TNM), jnp.float32), g[0:3]], axis=0)
    v = jnp.dot(uaug.T, p4_ref[...],
                preferred_element_type=jnp.float32)
    s = filt * h0j4 * v
    ss = _seg_sum128(s)
    m0 = ss[:, 0:C]
    m1_0 = ss[:, C:2 * C]
    m1_1 = ss[:, 2 * C:3 * C]
    m1_2 = ss[:, 3 * C:4 * C]

    ohc = (cls_ref[...][0:TN] == zc_ref[...]).astype(jnp.float32)
    h0c = jnp.dot(ohc, emb4_ref[...][:, 0:C],
                  preferred_element_type=jnp.float32)
    h0n = _silu(jnp.dot(m0, w0_ref[...], preferred_element_type=jnp.float32)
                + jnp.dot(h0c, u0_ref[...], preferred_element_type=jnp.float32)
                + b0_ref[...])
    gate = _silu(jnp.dot(m0, wg_ref[...], preferred_element_type=jnp.float32)
                 + bg_ref[...])
    w1 = w1_ref[...]
    h1n_0 = jnp.dot(m1_0, w1, preferred_element_type=jnp.float32) * gate
    h1n_1 = jnp.dot(m1_1, w1, preferred_element_type=jnp.float32) * gate
    h1n_2 = jnp.dot(m1_2, w1, preferred_element_type=jnp.float32) * gate
    hcf = jnp.concatenate([h0n, h1n_0, h1n_1, h1n_2], axis=1)
    hc_ref[...] = hcf
    lo = lax.bitcast_convert_type(
        hcf[:, 0:2 * C].astype(jnp.bfloat16).astype(jnp.float32), jnp.int32)
    hi = lax.bitcast_convert_type(
        hcf[:, 2 * C:4 * C].astype(jnp.bfloat16).astype(jnp.float32),
        jnp.int32)
    word = jnp.bitwise_or(hi, lax.shift_right_logical(lo, 16))
    hp_ref[...] = lax.bitcast_convert_type(word, jnp.float32)


def _layer1_kernel(geo_ref, hcj_ref, hc_ref, p4_ref,
                   wra_ref, w0_ref, u0_ref, b0_ref,
                   wro_ref, out_ref):
    g = geo_ref[...]
    filt = jnp.dot(g[3:3 + NB + 1].T, wra_ref[...],
                   preferred_element_type=jnp.float32)
    uaug = jnp.concatenate(
        [jnp.ones((1, TNM), jnp.float32), g[0:3]], axis=0)
    v = jnp.dot(uaug.T, p4_ref[...],
                preferred_element_type=jnp.float32)
    wi = lax.bitcast_convert_type(hcj_ref[...], jnp.int32)
    lo = lax.bitcast_convert_type(lax.shift_left(wi, 16), jnp.float32)
    hi = lax.bitcast_convert_type(
        jnp.bitwise_and(wi, jnp.int32(-65536)), jnp.float32)
    hcj = jnp.concatenate([lo, hi], axis=1)
    s = filt * hcj * v
    ss = _seg_sum128(s)
    m0 = (ss[:, 0:C] + ss[:, C:2 * C]
          + ss[:, 2 * C:3 * C] + ss[:, 3 * C:4 * C])

    h0c = hc_ref[...][:, 0:C]
    h0n = _silu(jnp.dot(m0, w0_ref[...], preferred_element_type=jnp.float32)
                + jnp.dot(h0c, u0_ref[...], preferred_element_type=jnp.float32)
                + b0_ref[...])
    out_ref[...] = jnp.dot(h0n, wro_ref[...], preferred_element_type=jnp.float32)


def _edge_spec(d):
    return pl.BlockSpec((TNM, d), lambda i: (i, 0))


def _atom_spec(d):
    return pl.BlockSpec((TN, d), lambda i: (i, 0))


def _full_spec(r, c):
    return pl.BlockSpec((r, c), lambda i: (0, 0))


def _plane_spec(p):
    return pl.BlockSpec((p, GR, 128), lambda i: (0, i, 0))


@jax.jit
def kernel(coordinate, atomic_number, neighbor, mask, emb_table, Wr, br,
           W0, U0, b0, W1, U1, Wg, bg, Wro, bro):
    f32 = jnp.float32
    coord2 = coordinate.reshape(BN, 3)
    zf = atomic_number.astype(f32).reshape(BN, 1)
    planes4 = jnp.concatenate([coord2.T, zf.T], axis=0).reshape(4 * BN)
    fidx_flat = (neighbor.astype(jnp.int32)
                 + (jnp.arange(B, dtype=jnp.int32) * N).reshape(B, 1, 1)
                 ).reshape(E)
    emb4 = jnp.concatenate([emb_table] * 4, axis=1)
    cls_row = jnp.arange(100, dtype=f32).reshape(1, 100)
    p4 = jnp.kron(jnp.eye(4, dtype=f32), jnp.ones((1, C), f32))
    wr_aug = jnp.concatenate([Wr, br[:, None, :]], axis=1)
    wrb = jnp.concatenate([wr_aug[0, :, 0:C]]
                          + [wr_aug[0, :, C:2 * C]] * 3, axis=1)
    wra = jnp.concatenate([wr_aug[1, :, 0:C]]
                          + [wr_aug[1, :, 2 * C:3 * C]] * 3, axis=1)

    e4 = _get_sc("static")(planes4, fidx_flat)

    cenT = jnp.repeat(coord2.T, M, axis=1)
    maskT = mask.astype(f32).reshape(1, E)
    geo_pl = pl.pallas_call(
        _geo_kernel,
        grid=(NBLK,),
        in_specs=[
            pl.BlockSpec((4, TNM), lambda i: (0, i)),
            pl.BlockSpec((3, TNM), lambda i: (0, i)),
            pl.BlockSpec((1, TNM), lambda i: (0, i)),
        ],
        out_specs=pl.BlockSpec((NG, TNM), lambda i: (0, i)),
        out_shape=jax.ShapeDtypeStruct((NG, E), f32),
    )(e4, cenT, maskT)

    zj = e4[3].reshape(E, 1)
    fidx = fidx_flat.reshape(NW, CH, G)
    hc = pl.pallas_call(
        _layer0_kernel,
        grid=(NBLK,),
        in_specs=[
            pl.BlockSpec((NG, TNM), lambda i: (0, i)),
            _edge_spec(1), _atom_spec(1),
            _full_spec(100, 4 * C), _full_spec(TNM, 100), _full_spec(4, 4 * C),
            _full_spec(NB + 1, 4 * C),
            _full_spec(C, C), _full_spec(C, C), _full_spec(1, C),
            _full_spec(C, C), _full_spec(C, C), _full_spec(1, C),
        ],
        out_specs=[_atom_spec(4 * C), _atom_spec(2 * C)],
        out_shape=[jax.ShapeDtypeStruct((BN, 4 * C), f32),
                   jax.ShapeDtypeStruct((BN, 2 * C), f32)],
        compiler_params=pltpu.CompilerParams(fuse_transposed_lhs_in_matmul=True),
    )(geo_pl, zj, zf, emb4,
      jnp.broadcast_to(cls_row, (TNM, 100)), p4, wrb,
      W0[0], U0[0], b0[0].reshape(1, C),
      W1[0], Wg[0], bg[0].reshape(1, C))
    hc, hp = hc

    hcj = _get_sc("rows")(hp, fidx)

    wro_p = jnp.concatenate([Wro.astype(f32), jnp.zeros((C, 7), f32)], axis=1)
    out8 = pl.pallas_call(
        _layer1_kernel,
        grid=(NBLK,),
        in_specs=[
            pl.BlockSpec((NG, TNM), lambda i: (0, i)),
            _edge_spec(2 * C), _atom_spec(4 * C),
            _full_spec(4, 4 * C), _full_spec(NB + 1, 4 * C),
            _full_spec(C, C), _full_spec(C, C), _full_spec(1, C),
            _full_spec(C, 8),
        ],
        out_specs=_atom_spec(8),
        out_shape=jax.ShapeDtypeStruct((BN, 8), f32),
        compiler_params=pltpu.CompilerParams(fuse_transposed_lhs_in_matmul=True),
    )(geo_pl, hcj, hc, p4, wra,
      W0[1], U0[1], b0[1].reshape(1, C), wro_p)

    return out8[:, 0:1].reshape(B, N, 1) + bro


# speedup vs baseline: 1.2355x; 1.2355x over previous
import functools

import jax
import jax.numpy as jnp
from jax import lax
from jax.experimental import pallas as pl
from jax.experimental.pallas import tpu as pltpu
from jax.experimental.pallas import tpu_sc as plsc

B, N, M, C, NB, L = 4, 2048, 48, 32, 16, 2
RC = 5.0
E = B * N * M
BN = B * N
TN = 128
TNM = TN * M
NBLK = BN // TN
ER = E // 128
GR = ER // NBLK
NG = NB + 4

NC, NS = 2, 16
NW = NC * NS
PW = E // NW
G = 128
CH = PW // G


def _sc_mesh():
    return plsc.VectorSubcoreMesh(core_axis_name="c", subcore_axis_name="s")


def _make_static_gather():
    @functools.partial(
        pl.kernel,
        mesh=_sc_mesh(),
        out_type=jax.ShapeDtypeStruct((4, E), jnp.float32),
        compiler_params=pltpu.CompilerParams(needs_layout_passes=False),
        scratch_types=[
            pltpu.VMEM((4 * BN,), jnp.float32),
            pltpu.VMEM((PW,), jnp.int32),
            pltpu.VMEM((4, PW), jnp.float32),
            pltpu.SemaphoreType.DMA,
        ],
    )
    def static_gather(tab_hbm, idx_hbm, out_hbm, tab_v, idx_v, out_v, sem):
        wid = lax.axis_index("s") * NC + lax.axis_index("c")
        base = wid * PW
        pltpu.sync_copy(tab_hbm, tab_v)
        pltpu.sync_copy(idx_hbm.at[pl.ds(base, PW)], idx_v)

        def body(i, carry):
            iv = idx_v[pl.ds(i * 16, 16)]
            for p in range(4):
                out_v[p, pl.ds(i * 16, 16)] = plsc.load_gather(
                    tab_v, [iv + p * BN])
            return carry

        lax.fori_loop(0, PW // 16, body, 0)
        for p in range(4):
            pltpu.sync_copy(out_v.at[p], out_hbm.at[p, pl.ds(base, PW)])

    return static_gather


def _make_row_gather(D, ne):
    pwl = ne // NW
    chl = pwl // G

    @functools.partial(
        pl.kernel,
        mesh=_sc_mesh(),
        out_type=jax.ShapeDtypeStruct((ne, D), jnp.float32),
        compiler_params=pltpu.CompilerParams(use_tc_tiling_on_sc=False),
        scratch_types=[
            pltpu.VMEM((chl, G), jnp.int32),
            pltpu.VMEM((G, D), jnp.float32),
            pltpu.SemaphoreType.DMA,
        ],
    )
    def row_gather(table_hbm, idx_hbm, out_hbm, idx_v, rows_v, sem):
        wid = lax.axis_index("s") * NC + lax.axis_index("c")
        base = wid * pwl
        pltpu.sync_copy(idx_hbm.at[wid], idx_v)

        def body(j, carry):
            pltpu.async_copy(table_hbm.at[idx_v.at[j]], rows_v, sem).wait()
            pltpu.sync_copy(rows_v, out_hbm.at[pl.ds(base + j * G, G)])
            return carry

        lax.fori_loop(0, chl, body, 0)

    return row_gather


_sc_cache = {}


def _get_sc(name):
    if name not in _sc_cache:
        _sc_cache[name] = (_make_static_gather() if name == "static"
                           else _make_row_gather(4 * C, E // 2))
    return _sc_cache[name]


def _silu(x):
    return x * (1.0 / (1.0 + jnp.exp(-x)))


def _geo_kernel(e4_ref, cen_ref, mask_ref, geo_ref):
    rx = e4_ref[0] - cen_ref[0]
    ry = e4_ref[1] - cen_ref[1]
    rz = e4_ref[2] - cen_ref[2]
    d = jnp.sqrt(rx * rx + ry * ry + rz * rz + 1e-12)
    inv = 1.0 / d
    geo_ref[0] = rx * inv
    geo_ref[1] = ry * inv
    geo_ref[2] = rz * inv
    dc = jnp.clip(d, 0.0, RC)
    env = 0.5 * (jnp.cos(dc * (jnp.pi / RC)) + 1.0) * (d < RC).astype(jnp.float32)
    maskf = mask_ref[0] * env
    geo_ref[NB + 3] = maskf
    th = d * (jnp.pi / RC)
    sinv = inv * maskf
    for k in range(NB):
        geo_ref[3 + k] = jnp.sin((k + 1.0) * th) * sinv


def _seg_sum128(x):
    return jnp.sum(x.reshape(TN, M, 4 * C), axis=1)


_TDN = (((0,), (0,)), ((), ()))


def _layer0_kernel(geo_ref, zj_ref, zc_ref, emb4_ref, cls_ref, p4_ref,
                   wrb_ref, w0_ref, u0_ref, b0_ref,
                   w1_ref, wg_ref, bg_ref, hc_ref):
    g = geo_ref[...]
    filt = jnp.dot(g[3:3 + NB + 1].T, wrb_ref[...],
                   preferred_element_type=jnp.float32)
    oh = (cls_ref[...] == zj_ref[...]).astype(jnp.float32)
    h0j4 = jnp.dot(oh, emb4_ref[...],
                   preferred_element_type=jnp.float32)
    uaug = jnp.concatenate(
        [jnp.ones((1, TNM), jnp.float32), g[0:3]], axis=0)
    v = jnp.dot(uaug.T, p4_ref[...],
                preferred_element_type=jnp.float32)
    s = filt * h0j4 * v
    ss = _seg_sum128(s)
    m0 = ss[:, 0:C]
    m1_0 = ss[:, C:2 * C]
    m1_1 = ss[:, 2 * C:3 * C]
    m1_2 = ss[:, 3 * C:4 * C]

    ohc = (cls_ref[...][0:TN] == zc_ref[...]).astype(jnp.float32)
    h0c = jnp.dot(ohc, emb4_ref[...][:, 0:C],
                  preferred_element_type=jnp.float32)
    h0n = _silu(jnp.dot(m0, w0_ref[...], preferred_element_type=jnp.float32)
                + jnp.dot(h0c, u0_ref[...], preferred_element_type=jnp.float32)
                + b0_ref[...])
    gate = _silu(jnp.dot(m0, wg_ref[...], preferred_element_type=jnp.float32)
                 + bg_ref[...])
    w1 = w1_ref[...]
    h1n_0 = jnp.dot(m1_0, w1, preferred_element_type=jnp.float32) * gate
    h1n_1 = jnp.dot(m1_1, w1, preferred_element_type=jnp.float32) * gate
    h1n_2 = jnp.dot(m1_2, w1, preferred_element_type=jnp.float32) * gate
    hc_ref[...] = jnp.concatenate([h0n, h1n_0, h1n_1, h1n_2], axis=1)


def _layer1_kernel(geo_ref, hcj_ref, hc_ref, p4_ref,
                   wra_ref, w0_ref, u0_ref, b0_ref,
                   wro_ref, out_ref):
    g = geo_ref[...]
    filt = jnp.dot(g[3:3 + NB + 1].T, wra_ref[...],
                   preferred_element_type=jnp.float32)
    uaug = jnp.concatenate(
        [jnp.ones((1, TNM), jnp.float32), g[0:3]], axis=0)
    v = jnp.dot(uaug.T, p4_ref[...],
                preferred_element_type=jnp.float32)
    s = filt * hcj_ref[...] * v
    ss = _seg_sum128(s)
    m0 = (ss[:, 0:C] + ss[:, C:2 * C]
          + ss[:, 2 * C:3 * C] + ss[:, 3 * C:4 * C])

    h0c = hc_ref[...][:, 0:C]
    h0n = _silu(jnp.dot(m0, w0_ref[...], preferred_element_type=jnp.float32)
                + jnp.dot(h0c, u0_ref[...], preferred_element_type=jnp.float32)
                + b0_ref[...])
    out_ref[...] = jnp.dot(h0n, wro_ref[...], preferred_element_type=jnp.float32)


def _edge_spec(d):
    return pl.BlockSpec((TNM, d), lambda i: (i, 0))


def _atom_spec(d):
    return pl.BlockSpec((TN, d), lambda i: (i, 0))


def _full_spec(r, c):
    return pl.BlockSpec((r, c), lambda i: (0, 0))


def _plane_spec(p):
    return pl.BlockSpec((p, GR, 128), lambda i: (0, i, 0))


@jax.jit
def kernel(coordinate, atomic_number, neighbor, mask, emb_table, Wr, br,
           W0, U0, b0, W1, U1, Wg, bg, Wro, bro):
    f32 = jnp.float32
    coord2 = coordinate.reshape(BN, 3)
    zf = atomic_number.astype(f32).reshape(BN, 1)
    planes4 = jnp.concatenate([coord2.T, zf.T], axis=0).reshape(4 * BN)
    fidx_flat = (neighbor.astype(jnp.int32)
                 + (jnp.arange(B, dtype=jnp.int32) * N).reshape(B, 1, 1)
                 ).reshape(E)
    emb4 = jnp.concatenate([emb_table] * 4, axis=1)
    cls_row = jnp.arange(100, dtype=f32).reshape(1, 100)
    p4 = jnp.kron(jnp.eye(4, dtype=f32), jnp.ones((1, C), f32))
    wr_aug = jnp.concatenate([Wr, br[:, None, :]], axis=1)
    wrb = jnp.concatenate([wr_aug[0, :, 0:C]]
                          + [wr_aug[0, :, C:2 * C]] * 3, axis=1)
    wra = jnp.concatenate([wr_aug[1, :, 0:C]]
                          + [wr_aug[1, :, 2 * C:3 * C]] * 3, axis=1)

    e4 = _get_sc("static")(planes4, fidx_flat)

    cenT = jnp.repeat(coord2.T, M, axis=1)
    maskT = mask.astype(f32).reshape(1, E)
    geo_pl = pl.pallas_call(
        _geo_kernel,
        grid=(NBLK,),
        in_specs=[
            pl.BlockSpec((4, TNM), lambda i: (0, i)),
            pl.BlockSpec((3, TNM), lambda i: (0, i)),
            pl.BlockSpec((1, TNM), lambda i: (0, i)),
        ],
        out_specs=pl.BlockSpec((NG, TNM), lambda i: (0, i)),
        out_shape=jax.ShapeDtypeStruct((NG, E), f32),
    )(e4, cenT, maskT)

    zj = e4[3].reshape(E, 1)
    fidx = fidx_flat.reshape(NW, CH, G)
    hc = pl.pallas_call(
        _layer0_kernel,
        grid=(NBLK,),
        in_specs=[
            pl.BlockSpec((NG, TNM), lambda i: (0, i)),
            _edge_spec(1), _atom_spec(1),
            _full_spec(100, 4 * C), _full_spec(TNM, 100), _full_spec(4, 4 * C),
            _full_spec(NB + 1, 4 * C),
            _full_spec(C, C), _full_spec(C, C), _full_spec(1, C),
            _full_spec(C, C), _full_spec(C, C), _full_spec(1, C),
        ],
        out_specs=_atom_spec(4 * C),
        out_shape=jax.ShapeDtypeStruct((BN, 4 * C), f32),
        compiler_params=pltpu.CompilerParams(fuse_transposed_lhs_in_matmul=True),
    )(geo_pl, zj, zf, emb4,
      jnp.broadcast_to(cls_row, (TNM, 100)), p4, wrb,
      W0[0], U0[0], b0[0].reshape(1, C),
      W1[0], Wg[0], bg[0].reshape(1, C))

    wro_p = jnp.concatenate([Wro.astype(f32), jnp.zeros((C, 7), f32)], axis=1)
    halves = []
    hcjs = [_get_sc("rows")(hc, fidx_flat[h * (E // 2):(h + 1) * (E // 2)]
                            .reshape(NW, CH // 2, G)) for h in (0, 1)]
    for h in (0, 1):
        off = h * (NBLK // 2)
        halves.append(pl.pallas_call(
            _layer1_kernel,
            grid=(NBLK // 2,),
            in_specs=[
                pl.BlockSpec((NG, TNM), lambda i, off=off: (0, i + off)),
                _edge_spec(4 * C),
                pl.BlockSpec((TN, 4 * C), lambda i, off=off: (i + off, 0)),
                _full_spec(4, 4 * C), _full_spec(NB + 1, 4 * C),
                _full_spec(C, C), _full_spec(C, C), _full_spec(1, C),
                _full_spec(C, 8),
            ],
            out_specs=_atom_spec(8),
            out_shape=jax.ShapeDtypeStruct((BN // 2, 8), f32),
            compiler_params=pltpu.CompilerParams(
                fuse_transposed_lhs_in_matmul=True),
        )(geo_pl, hcjs[h], hc, p4, wra,
          W0[1], U0[1], b0[1].reshape(1, C), wro_p))

    out8 = jnp.concatenate(halves, axis=0)
    return out8[:, 0:1].reshape(B, N, 1) + bro


# speedup vs baseline: 1.3225x; 1.0703x over previous
import functools

import jax
import jax.numpy as jnp
from jax import lax
from jax.experimental import pallas as pl
from jax.experimental.pallas import tpu as pltpu
from jax.experimental.pallas import tpu_sc as plsc

B, N, M, C, NB, L = 4, 2048, 48, 32, 16, 2
RC = 5.0
E = B * N * M
BN = B * N
TN = 128
TNM = TN * M
NBLK = BN // TN
ER = E // 128
GR = ER // NBLK
NG = NB + 4

NC, NS = 2, 16
NW = NC * NS
PW = E // NW
G = 128
CH = PW // G


def _sc_mesh():
    return plsc.VectorSubcoreMesh(core_axis_name="c", subcore_axis_name="s")


def _make_static_gather():
    @functools.partial(
        pl.kernel,
        mesh=_sc_mesh(),
        out_type=jax.ShapeDtypeStruct((4, E), jnp.float32),
        compiler_params=pltpu.CompilerParams(needs_layout_passes=False),
        scratch_types=[
            pltpu.VMEM((4 * BN,), jnp.float32),
            pltpu.VMEM((PW,), jnp.int32),
            pltpu.VMEM((4, PW), jnp.float32),
            pltpu.SemaphoreType.DMA,
        ],
    )
    def static_gather(tab_hbm, idx_hbm, out_hbm, tab_v, idx_v, out_v, sem):
        wid = lax.axis_index("s") * NC + lax.axis_index("c")
        base = wid * PW
        pltpu.sync_copy(tab_hbm, tab_v)
        pltpu.sync_copy(idx_hbm.at[pl.ds(base, PW)], idx_v)

        def body(i, carry):
            iv = idx_v[pl.ds(i * 16, 16)]
            for p in range(4):
                out_v[p, pl.ds(i * 16, 16)] = plsc.load_gather(
                    tab_v, [iv + p * BN])
            return carry

        lax.fori_loop(0, PW // 16, body, 0)
        for p in range(4):
            pltpu.sync_copy(out_v.at[p], out_hbm.at[p, pl.ds(base, PW)])

    return static_gather


def _make_row_gather(D, ne):
    pwl = ne // NW
    chl = pwl // G

    @functools.partial(
        pl.kernel,
        mesh=_sc_mesh(),
        out_type=jax.ShapeDtypeStruct((ne, D), jnp.float32),
        compiler_params=pltpu.CompilerParams(use_tc_tiling_on_sc=False),
        scratch_types=[
            pltpu.VMEM((chl, G), jnp.int32),
            pltpu.VMEM((G, D), jnp.float32),
            pltpu.SemaphoreType.DMA,
        ],
    )
    def row_gather(table_hbm, idx_hbm, out_hbm, idx_v, rows_v, sem):
        wid = lax.axis_index("s") * NC + lax.axis_index("c")
        base = wid * pwl
        pltpu.sync_copy(idx_hbm.at[wid], idx_v)

        def body(j, carry):
            pltpu.async_copy(table_hbm.at[idx_v.at[j]], rows_v, sem).wait()
            pltpu.sync_copy(rows_v, out_hbm.at[pl.ds(base + j * G, G)])
            return carry

        lax.fori_loop(0, chl, body, 0)

    return row_gather


_sc_cache = {}


def _get_sc(name):
    if name not in _sc_cache:
        _sc_cache[name] = (_make_static_gather() if name == "static"
                           else _make_row_gather(4 * C, E // 2))
    return _sc_cache[name]


def _silu(x):
    return x * (1.0 / (1.0 + jnp.exp(-x)))


def _geo_kernel(e4_ref, cen_ref, mask_ref, geo_ref):
    rx = e4_ref[0] - cen_ref[0]
    ry = e4_ref[1] - cen_ref[1]
    rz = e4_ref[2] - cen_ref[2]
    d = jnp.sqrt(rx * rx + ry * ry + rz * rz + 1e-12)
    inv = 1.0 / d
    geo_ref[0] = rx * inv
    geo_ref[1] = ry * inv
    geo_ref[2] = rz * inv
    dc = jnp.clip(d, 0.0, RC)
    env = 0.5 * (jnp.cos(dc * (jnp.pi / RC)) + 1.0) * (d < RC).astype(jnp.float32)
    maskf = mask_ref[0] * env
    geo_ref[NB + 3] = maskf
    th = d * (jnp.pi / RC)
    sinv = inv * maskf
    for k in range(NB):
        geo_ref[3 + k] = jnp.sin((k + 1.0) * th) * sinv


def _seg_sum128(x):
    return jnp.sum(x.reshape(TN, M, 4 * C), axis=1)


_TDN = (((0,), (0,)), ((), ()))


def _layer0_kernel(geo_ref, zj_ref, zc_ref, emb4_ref, cls_ref, p4_ref,
                   wrb_ref, w0_ref, u0_ref, b0_ref,
                   w1_ref, wg_ref, bg_ref, hc_ref):
    bf = jnp.bfloat16
    g = geo_ref[...]
    filt = jnp.dot(g[3:3 + NB + 1].astype(bf).T, wrb_ref[...].astype(bf),
                   preferred_element_type=jnp.float32)
    oh = (cls_ref[...] == zj_ref[...]).astype(bf)
    h0j4 = jnp.dot(oh, emb4_ref[...].astype(bf),
                   preferred_element_type=jnp.float32)
    uaug = jnp.concatenate(
        [jnp.ones((1, TNM), bf), g[0:3].astype(bf)], axis=0)
    v = jnp.dot(uaug.T, p4_ref[...].astype(bf),
                preferred_element_type=jnp.float32)
    s = filt * h0j4 * v
    ss = _seg_sum128(s)
    m0 = ss[:, 0:C]
    m1_0 = ss[:, C:2 * C]
    m1_1 = ss[:, 2 * C:3 * C]
    m1_2 = ss[:, 3 * C:4 * C]

    ohc = (cls_ref[...][0:TN] == zc_ref[...]).astype(jnp.float32)
    h0c = jnp.dot(ohc, emb4_ref[...][:, 0:C],
                  preferred_element_type=jnp.float32)
    h0n = _silu(jnp.dot(m0, w0_ref[...], preferred_element_type=jnp.float32)
                + jnp.dot(h0c, u0_ref[...], preferred_element_type=jnp.float32)
                + b0_ref[...])
    gate = _silu(jnp.dot(m0, wg_ref[...], preferred_element_type=jnp.float32)
                 + bg_ref[...])
    w1 = w1_ref[...]
    h1n_0 = jnp.dot(m1_0, w1, preferred_element_type=jnp.float32) * gate
    h1n_1 = jnp.dot(m1_1, w1, preferred_element_type=jnp.float32) * gate
    h1n_2 = jnp.dot(m1_2, w1, preferred_element_type=jnp.float32) * gate
    hc_ref[...] = jnp.concatenate([h0n, h1n_0, h1n_1, h1n_2], axis=1)


def _layer1_kernel(geo_ref, hcj_ref, hc_ref, p4_ref,
                   wra_ref, w0_ref, u0_ref, b0_ref,
                   wro_ref, out_ref):
    g = geo_ref[...]
    filt = jnp.dot(g[3:3 + NB + 1].T, wra_ref[...],
                   preferred_element_type=jnp.float32)
    uaug = jnp.concatenate(
        [jnp.ones((1, TNM), jnp.float32), g[0:3]], axis=0)
    v = jnp.dot(uaug.T, p4_ref[...],
                preferred_element_type=jnp.float32)
    s = filt * hcj_ref[...] * v
    ss = _seg_sum128(s)
    m0 = (ss[:, 0:C] + ss[:, C:2 * C]
          + ss[:, 2 * C:3 * C] + ss[:, 3 * C:4 * C])

    h0c = hc_ref[...][:, 0:C]
    h0n = _silu(jnp.dot(m0, w0_ref[...], preferred_element_type=jnp.float32)
                + jnp.dot(h0c, u0_ref[...], preferred_element_type=jnp.float32)
                + b0_ref[...])
    out_ref[...] = jnp.dot(h0n, wro_ref[...], preferred_element_type=jnp.float32)


def _edge_spec(d):
    return pl.BlockSpec((TNM, d), lambda i: (i, 0))


def _atom_spec(d):
    return pl.BlockSpec((TN, d), lambda i: (i, 0))


def _full_spec(r, c):
    return pl.BlockSpec((r, c), lambda i: (0, 0))


def _plane_spec(p):
    return pl.BlockSpec((p, GR, 128), lambda i: (0, i, 0))


@jax.jit
def kernel(coordinate, atomic_number, neighbor, mask, emb_table, Wr, br,
           W0, U0, b0, W1, U1, Wg, bg, Wro, bro):
    f32 = jnp.float32
    coord2 = coordinate.reshape(BN, 3)
    zf = atomic_number.astype(f32).reshape(BN, 1)
    planes4 = jnp.concatenate([coord2.T, zf.T], axis=0).reshape(4 * BN)
    fidx_flat = (neighbor.astype(jnp.int32)
                 + (jnp.arange(B, dtype=jnp.int32) * N).reshape(B, 1, 1)
                 ).reshape(E)
    emb4 = jnp.concatenate([emb_table] * 4, axis=1)
    cls_row = jnp.arange(100, dtype=f32).reshape(1, 100)
    p4 = jnp.kron(jnp.eye(4, dtype=f32), jnp.ones((1, C), f32))
    wr_aug = jnp.concatenate([Wr, br[:, None, :]], axis=1)
    wrb = jnp.concatenate([wr_aug[0, :, 0:C]]
                          + [wr_aug[0, :, C:2 * C]] * 3, axis=1)
    wra = jnp.concatenate([wr_aug[1, :, 0:C]]
                          + [wr_aug[1, :, 2 * C:3 * C]] * 3, axis=1)

    e4 = _get_sc("static")(planes4, fidx_flat)

    cenT = jnp.repeat(coord2.T, M, axis=1)
    maskT = mask.astype(f32).reshape(1, E)
    geo_pl = pl.pallas_call(
        _geo_kernel,
        grid=(NBLK,),
        in_specs=[
            pl.BlockSpec((4, TNM), lambda i: (0, i)),
            pl.BlockSpec((3, TNM), lambda i: (0, i)),
            pl.BlockSpec((1, TNM), lambda i: (0, i)),
        ],
        out_specs=pl.BlockSpec((NG, TNM), lambda i: (0, i)),
        out_shape=jax.ShapeDtypeStruct((NG, E), f32),
    )(e4, cenT, maskT)

    zj = e4[3].reshape(E, 1)
    fidx = fidx_flat.reshape(NW, CH, G)
    hc = pl.pallas_call(
        _layer0_kernel,
        grid=(NBLK,),
        in_specs=[
            pl.BlockSpec((NG, TNM), lambda i: (0, i)),
            _edge_spec(1), _atom_spec(1),
            _full_spec(100, 4 * C), _full_spec(TNM, 100), _full_spec(4, 4 * C),
            _full_spec(NB + 1, 4 * C),
            _full_spec(C, C), _full_spec(C, C), _full_spec(1, C),
            _full_spec(C, C), _full_spec(C, C), _full_spec(1, C),
        ],
        out_specs=_atom_spec(4 * C),
        out_shape=jax.ShapeDtypeStruct((BN, 4 * C), f32),
        compiler_params=pltpu.CompilerParams(fuse_transposed_lhs_in_matmul=True),
    )(geo_pl, zj, zf, emb4,
      jnp.broadcast_to(cls_row, (TNM, 100)), p4, wrb,
      W0[0], U0[0], b0[0].reshape(1, C),
      W1[0], Wg[0], bg[0].reshape(1, C))

    wro_p = jnp.concatenate([Wro.astype(f32), jnp.zeros((C, 7), f32)], axis=1)
    halves = []
    hcjs = [_get_sc("rows")(hc, fidx_flat[h * (E // 2):(h + 1) * (E // 2)]
                            .reshape(NW, CH // 2, G)) for h in (0, 1)]
    for h in (0, 1):
        off = h * (NBLK // 2)
        halves.append(pl.pallas_call(
            _layer1_kernel,
            grid=(NBLK // 2,),
            in_specs=[
                pl.BlockSpec((NG, TNM), lambda i, off=off: (0, i + off)),
                _edge_spec(4 * C),
                pl.BlockSpec((TN, 4 * C), lambda i, off=off: (i + off, 0)),
                _full_spec(4, 4 * C), _full_spec(NB + 1, 4 * C),
                _full_spec(C, C), _full_spec(C, C), _full_spec(1, C),
                _full_spec(C, 8),
            ],
            out_specs=_atom_spec(8),
            out_shape=jax.ShapeDtypeStruct((BN // 2, 8), f32),
            compiler_params=pltpu.CompilerParams(
                fuse_transposed_lhs_in_matmul=True),
        )(geo_pl, hcjs[h], hc, p4, wra,
          W0[1], U0[1], b0[1].reshape(1, C), wro_p))

    out8 = jnp.concatenate(halves, axis=0)
    return out8[:, 0:1].reshape(B, N, 1) + bro


# speedup vs baseline: 1.3543x; 1.0241x over previous
import functools

import jax
import jax.numpy as jnp
from jax import lax
from jax.experimental import pallas as pl
from jax.experimental.pallas import tpu as pltpu
from jax.experimental.pallas import tpu_sc as plsc

B, N, M, C, NB, L = 4, 2048, 48, 32, 16, 2
RC = 5.0
E = B * N * M
BN = B * N
TN = 128
TNM = TN * M
NBLK = BN // TN
ER = E // 128
GR = ER // NBLK
NG = NB + 4

NC, NS = 2, 16
NW = NC * NS
PW = E // NW
G = 128
CH = PW // G


def _sc_mesh():
    return plsc.VectorSubcoreMesh(core_axis_name="c", subcore_axis_name="s")


def _make_static_gather():
    @functools.partial(
        pl.kernel,
        mesh=_sc_mesh(),
        out_type=jax.ShapeDtypeStruct((4, E), jnp.float32),
        compiler_params=pltpu.CompilerParams(needs_layout_passes=False),
        scratch_types=[
            pltpu.VMEM((4 * BN,), jnp.float32),
            pltpu.VMEM((PW,), jnp.int32),
            pltpu.VMEM((4, PW), jnp.float32),
            pltpu.SemaphoreType.DMA,
        ],
    )
    def static_gather(tab_hbm, idx_hbm, out_hbm, tab_v, idx_v, out_v, sem):
        wid = lax.axis_index("s") * NC + lax.axis_index("c")
        base = wid * PW
        pltpu.sync_copy(tab_hbm, tab_v)
        pltpu.sync_copy(idx_hbm.at[pl.ds(base, PW)], idx_v)

        def body(i, carry):
            iv = idx_v[pl.ds(i * 16, 16)]
            for p in range(4):
                out_v[p, pl.ds(i * 16, 16)] = plsc.load_gather(
                    tab_v, [iv + p * BN])
            return carry

        lax.fori_loop(0, PW // 16, body, 0)
        for p in range(4):
            pltpu.sync_copy(out_v.at[p], out_hbm.at[p, pl.ds(base, PW)])

    return static_gather


def _make_row_gather(D, ne):
    pwl = ne // NW
    chl = pwl // G

    @functools.partial(
        pl.kernel,
        mesh=_sc_mesh(),
        out_type=jax.ShapeDtypeStruct((ne, D), jnp.float32),
        compiler_params=pltpu.CompilerParams(use_tc_tiling_on_sc=False),
        scratch_types=[
            pltpu.VMEM((chl, G), jnp.int32),
            pltpu.VMEM((G, D), jnp.float32),
            pltpu.SemaphoreType.DMA,
        ],
    )
    def row_gather(table_hbm, idx_hbm, out_hbm, idx_v, rows_v, sem):
        wid = lax.axis_index("s") * NC + lax.axis_index("c")
        base = wid * pwl
        pltpu.sync_copy(idx_hbm.at[wid], idx_v)

        def body(j, carry):
            pltpu.async_copy(table_hbm.at[idx_v.at[j]], rows_v, sem).wait()
            pltpu.sync_copy(rows_v, out_hbm.at[pl.ds(base + j * G, G)])
            return carry

        lax.fori_loop(0, chl, body, 0)

    return row_gather


_sc_cache = {}


def _get_sc(name):
    if name not in _sc_cache:
        _sc_cache[name] = (_make_static_gather() if name == "static"
                           else _make_row_gather(4 * C, E // 2))
    return _sc_cache[name]


def _silu(x):
    return x * (1.0 / (1.0 + jnp.exp(-x)))


def _geo_kernel(e4_ref, cen_ref, mask_ref, geo_ref):
    rx = e4_ref[0] - cen_ref[0]
    ry = e4_ref[1] - cen_ref[1]
    rz = e4_ref[2] - cen_ref[2]
    d = jnp.sqrt(rx * rx + ry * ry + rz * rz + 1e-12)
    inv = 1.0 / d
    geo_ref[0] = rx * inv
    geo_ref[1] = ry * inv
    geo_ref[2] = rz * inv
    dc = jnp.clip(d, 0.0, RC)
    env = 0.5 * (jnp.cos(dc * (jnp.pi / RC)) + 1.0) * (d < RC).astype(jnp.float32)
    maskf = mask_ref[0] * env
    geo_ref[NB + 3] = maskf
    th = d * (jnp.pi / RC)
    sinv = inv * maskf
    for k in range(NB):
        geo_ref[3 + k] = jnp.sin((k + 1.0) * th) * sinv


def _seg_sum128(x):
    return jnp.sum(x.reshape(TN, M, 4 * C), axis=1)


_TDN = (((0,), (0,)), ((), ()))


def _layer0_kernel(geo_ref, zj_ref, zc_ref, emb4_ref, cls_ref, p4_ref,
                   wrb_ref, w0_ref, u0_ref, b0_ref,
                   w1_ref, wg_ref, bg_ref, hc_ref):
    bf = jnp.bfloat16
    g = geo_ref[...]
    filt = jnp.dot(g[3:3 + NB + 1].astype(bf).T, wrb_ref[...].astype(bf),
                   preferred_element_type=jnp.float32)
    oh = (cls_ref[...] == zj_ref[...]).astype(bf)
    h0j4 = jnp.dot(oh, emb4_ref[...].astype(bf),
                   preferred_element_type=jnp.float32)
    uaug = jnp.concatenate(
        [jnp.ones((1, TNM), bf), g[0:3].astype(bf)], axis=0)
    v = jnp.dot(uaug.T, p4_ref[...].astype(bf),
                preferred_element_type=jnp.float32)
    s = filt * h0j4 * v
    ss = _seg_sum128(s)
    m0 = ss[:, 0:C]
    m1_0 = ss[:, C:2 * C]
    m1_1 = ss[:, 2 * C:3 * C]
    m1_2 = ss[:, 3 * C:4 * C]

    ohc = (cls_ref[...][0:TN] == zc_ref[...]).astype(jnp.float32)
    h0c = jnp.dot(ohc, emb4_ref[...][:, 0:C],
                  preferred_element_type=jnp.float32)
    h0n = _silu(jnp.dot(m0, w0_ref[...], preferred_element_type=jnp.float32)
                + jnp.dot(h0c, u0_ref[...], preferred_element_type=jnp.float32)
                + b0_ref[...])
    gate = _silu(jnp.dot(m0, wg_ref[...], preferred_element_type=jnp.float32)
                 + bg_ref[...])
    w1 = w1_ref[...]
    h1n_0 = jnp.dot(m1_0, w1, preferred_element_type=jnp.float32) * gate
    h1n_1 = jnp.dot(m1_1, w1, preferred_element_type=jnp.float32) * gate
    h1n_2 = jnp.dot(m1_2, w1, preferred_element_type=jnp.float32) * gate
    hc_ref[...] = jnp.concatenate([h0n, h1n_0, h1n_1, h1n_2], axis=1)


def _layer1_kernel(geo_ref, hcj_ref, hc_ref, p4_ref,
                   wra_ref, w0_ref, u0_ref, b0_ref,
                   wro_ref, out_ref):
    bf = jnp.bfloat16
    g = geo_ref[...]
    filt = jnp.dot(g[3:3 + NB + 1].astype(bf).T, wra_ref[...].astype(bf),
                   preferred_element_type=jnp.float32)
    uaug = jnp.concatenate(
        [jnp.ones((1, TNM), bf), g[0:3].astype(bf)], axis=0)
    v = jnp.dot(uaug.T, p4_ref[...].astype(bf),
                preferred_element_type=jnp.float32)
    s = filt * hcj_ref[...] * v
    ss = _seg_sum128(s)
    m0 = (ss[:, 0:C] + ss[:, C:2 * C]
          + ss[:, 2 * C:3 * C] + ss[:, 3 * C:4 * C])

    h0c = hc_ref[...][:, 0:C]
    h0n = _silu(jnp.dot(m0, w0_ref[...], preferred_element_type=jnp.float32)
                + jnp.dot(h0c, u0_ref[...], preferred_element_type=jnp.float32)
                + b0_ref[...])
    out_ref[...] = jnp.dot(h0n, wro_ref[...], preferred_element_type=jnp.float32)


def _edge_spec(d):
    return pl.BlockSpec((TNM, d), lambda i: (i, 0))


def _atom_spec(d):
    return pl.BlockSpec((TN, d), lambda i: (i, 0))


def _full_spec(r, c):
    return pl.BlockSpec((r, c), lambda i: (0, 0))


def _plane_spec(p):
    return pl.BlockSpec((p, GR, 128), lambda i: (0, i, 0))


@jax.jit
def kernel(coordinate, atomic_number, neighbor, mask, emb_table, Wr, br,
           W0, U0, b0, W1, U1, Wg, bg, Wro, bro):
    f32 = jnp.float32
    coord2 = coordinate.reshape(BN, 3)
    zf = atomic_number.astype(f32).reshape(BN, 1)
    planes4 = jnp.concatenate([coord2.T, zf.T], axis=0).reshape(4 * BN)
    fidx_flat = (neighbor.astype(jnp.int32)
                 + (jnp.arange(B, dtype=jnp.int32) * N).reshape(B, 1, 1)
                 ).reshape(E)
    emb4 = jnp.concatenate([emb_table] * 4, axis=1)
    cls_row = jnp.arange(100, dtype=f32).reshape(1, 100)
    p4 = jnp.kron(jnp.eye(4, dtype=f32), jnp.ones((1, C), f32))
    wr_aug = jnp.concatenate([Wr, br[:, None, :]], axis=1)
    wrb = jnp.concatenate([wr_aug[0, :, 0:C]]
                          + [wr_aug[0, :, C:2 * C]] * 3, axis=1)
    wra = jnp.concatenate([wr_aug[1, :, 0:C]]
                          + [wr_aug[1, :, 2 * C:3 * C]] * 3, axis=1)

    e4 = _get_sc("static")(planes4, fidx_flat)

    cenT = jnp.repeat(coord2.T, M, axis=1)
    maskT = mask.astype(f32).reshape(1, E)
    geo_pl = pl.pallas_call(
        _geo_kernel,
        grid=(NBLK,),
        in_specs=[
            pl.BlockSpec((4, TNM), lambda i: (0, i)),
            pl.BlockSpec((3, TNM), lambda i: (0, i)),
            pl.BlockSpec((1, TNM), lambda i: (0, i)),
        ],
        out_specs=pl.BlockSpec((NG, TNM), lambda i: (0, i)),
        out_shape=jax.ShapeDtypeStruct((NG, E), f32),
    )(e4, cenT, maskT)

    zj = e4[3].reshape(E, 1)
    fidx = fidx_flat.reshape(NW, CH, G)
    hc = pl.pallas_call(
        _layer0_kernel,
        grid=(NBLK,),
        in_specs=[
            pl.BlockSpec((NG, TNM), lambda i: (0, i)),
            _edge_spec(1), _atom_spec(1),
            _full_spec(100, 4 * C), _full_spec(TNM, 100), _full_spec(4, 4 * C),
            _full_spec(NB + 1, 4 * C),
            _full_spec(C, C), _full_spec(C, C), _full_spec(1, C),
            _full_spec(C, C), _full_spec(C, C), _full_spec(1, C),
        ],
        out_specs=_atom_spec(4 * C),
        out_shape=jax.ShapeDtypeStruct((BN, 4 * C), f32),
        compiler_params=pltpu.CompilerParams(fuse_transposed_lhs_in_matmul=True),
    )(geo_pl, zj, zf, emb4,
      jnp.broadcast_to(cls_row, (TNM, 100)), p4, wrb,
      W0[0], U0[0], b0[0].reshape(1, C),
      W1[0], Wg[0], bg[0].reshape(1, C))

    wro_p = jnp.concatenate([Wro.astype(f32), jnp.zeros((C, 7), f32)], axis=1)
    halves = []
    hcjs = [_get_sc("rows")(hc, fidx_flat[h * (E // 2):(h + 1) * (E // 2)]
                            .reshape(NW, CH // 2, G)) for h in (0, 1)]
    for h in (0, 1):
        off = h * (NBLK // 2)
        halves.append(pl.pallas_call(
            _layer1_kernel,
            grid=(NBLK // 2,),
            in_specs=[
                pl.BlockSpec((NG, TNM), lambda i, off=off: (0, i + off)),
                _edge_spec(4 * C),
                pl.BlockSpec((TN, 4 * C), lambda i, off=off: (i + off, 0)),
                _full_spec(4, 4 * C), _full_spec(NB + 1, 4 * C),
                _full_spec(C, C), _full_spec(C, C), _full_spec(1, C),
                _full_spec(C, 8),
            ],
            out_specs=_atom_spec(8),
            out_shape=jax.ShapeDtypeStruct((BN // 2, 8), f32),
            compiler_params=pltpu.CompilerParams(
                fuse_transposed_lhs_in_matmul=True),
        )(geo_pl, hcjs[h], hc, p4, wra,
          W0[1], U0[1], b0[1].reshape(1, C), wro_p))

    out8 = jnp.concatenate(halves, axis=0)
    return out8[:, 0:1].reshape(B, N, 1) + bro
